# baseline (speedup 1.0000x reference)
import functools

import jax
import jax.numpy as jnp
from jax import lax
from jax.experimental import pallas as pl
from jax.experimental.pallas import tpu as pltpu
from jax.experimental.pallas import tpu_sc as plsc

_NC = 2
_NS = 16
_NW = _NC * _NS
_GSZ = 128
_NB = 3


@functools.partial(jax.jit, static_argnames=("b", "h", "d"))
def _sc_embedding_gather(idx_flat, weight, *, b, h, d):
    n_dd = d // 8
    n_bb = b // _GSZ
    groups = h * n_bb
    g_w = groups // _NW
    assert groups % _NW == 0 and g_w >= 3 * _NB
    n_steady = ((g_w - 2 * _NB) // _NB) * _NB
    tail_start = _NB + n_steady

    mesh = plsc.VectorSubcoreMesh(core_axis_name="c", subcore_axis_name="s")

    scratch = (
        [pltpu.VMEM((_GSZ,), jnp.int32) for _ in range(_NB)]
        + [pltpu.VMEM((_GSZ, d), jnp.float32) for _ in range(_NB)]
        + [pltpu.VMEM((d, _GSZ + 1), jnp.float32) for _ in range(_NB)]
        + [pltpu.SemaphoreType.DMA for _ in range(3 * _NB)]
    )

    @functools.partial(
        pl.kernel,
        mesh=mesh,
        out_type=jax.ShapeDtypeStruct((h, n_dd, n_bb, 8, _GSZ), jnp.float32),
        scratch_types=scratch,
        compiler_params=pltpu.CompilerParams(use_tc_tiling_on_sc=False, needs_layout_passes=False),
    )
    def k(idx_hbm, w_hbm, out_hbm, *sc):
        idx_bufs = sc[0:_NB]
        row_bufs = sc[_NB:2 * _NB]
        tr_bufs = sc[2 * _NB:3 * _NB]
        sem_i = sc[3 * _NB:4 * _NB]
        sem_g = sc[4 * _NB:5 * _NB]
        sem_o = sc[5 * _NB:6 * _NB]

        wid = lax.axis_index("s") * _NC + lax.axis_index("c")
        g_base = wid * g_w
        lanes = lax.iota(jnp.int32, 16)

        def idx_copy(g, bf):
            src = idx_hbm.at[pl.ds(g * _GSZ, _GSZ)]
            return pltpu.make_async_copy(src, idx_bufs[bf], sem_i[bf])

        def gather_copy(bf):
            return pltpu.make_async_copy(
                w_hbm.at[idx_bufs[bf]], row_bufs[bf], sem_g[bf])

        def tile_copy(g, bf, dd):
            h_g = g // n_bb
            bb_g = g % n_bb
            dst = out_hbm.at[h_g, dd, bb_g]
            return pltpu.make_async_copy(
                tr_bufs[bf].at[pl.ds(8 * dd, 8), pl.ds(0, _GSZ)],
                dst, sem_o[bf])

        def drain_tiles(bf):
            for dd in range(n_dd):
                pltpu.make_async_copy(
                    tr_bufs[bf].at[pl.ds(8 * dd, 8), pl.ds(0, _GSZ)],
                    out_hbm.at[0, dd, 0], sem_o[bf]).wait()

        def transpose(bf):
            rows = row_bufs[bf]
            tr = tr_bufs[bf]
            for bb8 in range(_GSZ):
                for d0 in range(0, d, 16):
                    vals = rows[bb8, pl.ds(d0, 16)]
                    plsc.store_scatter(
                        tr, [d0 + lanes, jnp.full((16,), bb8, jnp.int32)],
                        vals)

        def body(g, bf, *, launch, wait_o, load):
            bn = (bf + 1) % _NB
            if launch:
                idx_copy(0, bn).wait()
                gather_copy(bn).start()
            gather_copy(bf).wait()
            if wait_o:
                drain_tiles(bf)
            transpose(bf)
            for dd in range(n_dd):
                tile_copy(g, bf, dd).start()
            if load:
                idx_copy(g + 2, (bf + 2) % _NB).start()

        idx_copy(g_base + 0, 0).start()
        idx_copy(g_base + 1, 1).start()
        idx_copy(0, 0).wait()
        gather_copy(0).start()
        for t in range(_NB):
            body(g_base + t, t,
                 launch=(t + 1 < g_w), wait_o=False, load=(t + 2 < g_w))

        def steady(s, carry):
            g = g_base + _NB + s * _NB
            for j in range(_NB):
                body(g + j, j, launch=True, wait_o=True, load=True)
            return carry

        lax.fori_loop(0, n_steady // _NB, steady, 0)

        for t in range(tail_start, g_w):
            body(g_base + t, t % _NB,
                 launch=(t + 1 < g_w), wait_o=True, load=(t + 2 < g_w))
        for bf in range(_NB):
            drain_tiles(bf)

    return k(idx_flat, weight)


def kernel(idx, weight):
    b, h = idx.shape
    v, d = weight.shape
    idx_flat = idx.T.reshape(b * h).astype(jnp.int32)
    out5 = _sc_embedding_gather(idx_flat, weight, b=b, h=h, d=d)
    return out5.transpose(2, 4, 0, 1, 3).reshape(b, h, d)


# speedup vs baseline: 1.6750x; 1.6750x over previous
import functools

import jax
import jax.numpy as jnp
from jax import lax
from jax.experimental import pallas as pl
from jax.experimental.pallas import tpu as pltpu
from jax.experimental.pallas import tpu_sc as plsc

_NC = 2
_NS = 16
_NW = _NC * _NS
_GSZ = 128
_NB = 3


@functools.partial(jax.jit, static_argnames=("b", "h", "d"))
def _sc_embedding_gather(idx_flat, weight, *, b, h, d):
    n_dd = d // 8
    n_bb = b // _GSZ
    groups = h * n_bb
    g_w = groups // _NW
    assert groups % _NW == 0 and g_w >= 3 * _NB
    n_steady = ((g_w - 2 * _NB) // _NB) * _NB
    tail_start = _NB + n_steady

    mesh = plsc.VectorSubcoreMesh(core_axis_name="c", subcore_axis_name="s")

    scratch = (
        [pltpu.VMEM((_GSZ,), jnp.int32) for _ in range(_NB)]
        + [pltpu.VMEM((_GSZ, d), jnp.float32) for _ in range(_NB)]
        + [pltpu.VMEM((d, _GSZ + 1), jnp.float32) for _ in range(_NB)]
        + [pltpu.SemaphoreType.DMA for _ in range(3 * _NB)]
    )

    @functools.partial(
        pl.kernel,
        mesh=mesh,
        out_type=jax.ShapeDtypeStruct((h, n_dd, n_bb, 8, _GSZ), jnp.float32),
        scratch_types=scratch,
        compiler_params=pltpu.CompilerParams(use_tc_tiling_on_sc=False, needs_layout_passes=False),
    )
    def k(idx_hbm, w_hbm, out_hbm, *sc):
        idx_bufs = sc[0:_NB]
        row_bufs = sc[_NB:2 * _NB]
        tr_bufs = sc[2 * _NB:3 * _NB]
        sem_i = sc[3 * _NB:4 * _NB]
        sem_g = sc[4 * _NB:5 * _NB]
        sem_o = sc[5 * _NB:6 * _NB]

        wid = lax.axis_index("s") * _NC + lax.axis_index("c")
        g_base = wid * g_w
        lanes = lax.iota(jnp.int32, 16)

        def idx_copy(g, bf):
            src = idx_hbm.at[pl.ds(g * _GSZ, _GSZ)]
            return pltpu.make_async_copy(src, idx_bufs[bf], sem_i[bf])

        def gather_copy(bf):
            return pltpu.make_async_copy(
                w_hbm.at[idx_bufs[bf]], row_bufs[bf], sem_g[bf])

        def tile_copy(g, bf, dd):
            h_g = g // n_bb
            bb_g = g % n_bb
            dst = out_hbm.at[h_g, dd, bb_g]
            return pltpu.make_async_copy(
                tr_bufs[bf].at[pl.ds(8 * dd, 8), pl.ds(0, _GSZ)],
                dst, sem_o[bf])

        def drain_tiles(bf):
            for dd in range(n_dd):
                pltpu.make_async_copy(
                    tr_bufs[bf].at[pl.ds(8 * dd, 8), pl.ds(0, _GSZ)],
                    out_hbm.at[0, dd, 0], sem_o[bf]).wait()

        def transpose(bf):
            rows = row_bufs[bf]
            tr = tr_bufs[bf]

            @plsc.parallel_loop(0, _GSZ, unroll=8)
            def _t(bb8):
                bvec = jnp.full((16,), bb8, jnp.int32)
                for d0 in range(0, d, 16):
                    vals = rows[bb8, pl.ds(d0, 16)]
                    plsc.store_scatter(tr, [d0 + lanes, bvec], vals)

        def body(g, bf, *, launch, wait_o, load):
            bn = (bf + 1) % _NB
            if launch:
                idx_copy(0, bn).wait()
                gather_copy(bn).start()
            gather_copy(bf).wait()
            if wait_o:
                drain_tiles(bf)
            transpose(bf)
            for dd in range(n_dd):
                tile_copy(g, bf, dd).start()
            if load:
                idx_copy(g + 2, (bf + 2) % _NB).start()

        idx_copy(g_base + 0, 0).start()
        idx_copy(g_base + 1, 1).start()
        idx_copy(0, 0).wait()
        gather_copy(0).start()
        for t in range(_NB):
            body(g_base + t, t,
                 launch=(t + 1 < g_w), wait_o=False, load=(t + 2 < g_w))

        def steady(s, carry):
            g = g_base + _NB + s * _NB
            for j in range(_NB):
                body(g + j, j, launch=True, wait_o=True, load=True)
            return carry

        lax.fori_loop(0, n_steady // _NB, steady, 0)

        for t in range(tail_start, g_w):
            body(g_base + t, t % _NB,
                 launch=(t + 1 < g_w), wait_o=True, load=(t + 2 < g_w))
        for bf in range(_NB):
            drain_tiles(bf)

    return k(idx_flat, weight)


def kernel(idx, weight):
    b, h = idx.shape
    v, d = weight.shape
    idx_flat = idx.T.reshape(b * h).astype(jnp.int32)
    out5 = _sc_embedding_gather(idx_flat, weight, b=b, h=h, d=d)
    return out5.transpose(2, 4, 0, 1, 3).reshape(b, h, d)


# speedup vs baseline: 2.3843x; 1.4235x over previous
import functools

import jax
import jax.numpy as jnp
from jax import lax
from jax.experimental import pallas as pl
from jax.experimental.pallas import tpu as pltpu
from jax.experimental.pallas import tpu_sc as plsc

_NC = 2
_NS = 16
_NW = _NC * _NS
_GSZ = 128
_SG = 8
_CH = _GSZ * _SG


@functools.partial(jax.jit, static_argnames=("b", "h", "d"))
def _sc_embedding_gather(idx_flat, weight, *, b, h, d):
    n_dd = d // 8
    n_bb = b // _GSZ
    groups = h * n_bb
    sgroups = groups // _SG
    s_w = sgroups // _NW
    assert groups % (_SG * _NW) == 0 and s_w >= 6 and n_bb % _SG == 0
    n_steady = ((s_w - 4) // 2) * 2
    tail_start = 1 + n_steady

    mesh = plsc.VectorSubcoreMesh(core_axis_name="c", subcore_axis_name="s")

    scratch = (
        [pltpu.VMEM((_CH,), jnp.int32) for _ in range(2)]
        + [pltpu.VMEM((_CH, d), jnp.float32) for _ in range(2)]
        + [pltpu.VMEM((d, _GSZ + 1), jnp.float32) for _ in range(2)]
        + [pltpu.SemaphoreType.DMA for _ in range(6)]
    )

    @functools.partial(
        pl.kernel,
        mesh=mesh,
        out_type=jax.ShapeDtypeStruct((h, n_dd, n_bb, 8, _GSZ), jnp.float32),
        scratch_types=scratch,
        compiler_params=pltpu.CompilerParams(
            use_tc_tiling_on_sc=False, needs_layout_passes=False),
    )
    def k(idx_hbm, w_hbm, out_hbm, *sc):
        idx_bufs = sc[0:2]
        row_bufs = sc[2:4]
        tr_bufs = sc[4:6]
        sem_i = sc[6:8]
        sem_g = sc[8:10]
        sem_o = sc[10:12]

        wid = lax.axis_index("s") * _NC + lax.axis_index("c")
        s_base = wid * s_w
        lanes = lax.iota(jnp.int32, 16)

        def idx_copy(sg, bf):
            src = idx_hbm.at[pl.ds(sg * _CH, _CH)]
            return pltpu.make_async_copy(src, idx_bufs[bf], sem_i[bf])

        def gather_copy(bf):
            return pltpu.make_async_copy(
                w_hbm.at[idx_bufs[bf]], row_bufs[bf], sem_g[bf])

        def tile_copy(g, tb, dd):
            h_g = g // n_bb
            bb_g = g % n_bb
            dst = out_hbm.at[h_g, dd, bb_g]
            return pltpu.make_async_copy(
                tr_bufs[tb].at[pl.ds(8 * dd, 8), pl.ds(0, _GSZ)],
                dst, sem_o[tb])

        def drain_tiles(tb):
            for dd in range(n_dd):
                pltpu.make_async_copy(
                    tr_bufs[tb].at[pl.ds(8 * dd, 8), pl.ds(0, _GSZ)],
                    out_hbm.at[0, dd, 0], sem_o[tb]).wait()

        def transpose(bf, j, tb):
            rows = row_bufs[bf]
            tr = tr_bufs[tb]

            @plsc.parallel_loop(0, _GSZ, unroll=8)
            def _t(bb8):
                bvec = jnp.full((16,), bb8, jnp.int32)
                for d0 in range(0, d, 16):
                    vals = rows[j * _GSZ + bb8, pl.ds(d0, 16)]
                    plsc.store_scatter(tr, [d0 + lanes, bvec], vals)

        def body(sg, bf, *, launch, load, first):
            if launch:
                idx_copy(0, 1 - bf).wait()
                gather_copy(1 - bf).start()
            gather_copy(bf).wait()
            for j in range(_SG):
                tb = j % 2
                if not (first and j < 2):
                    drain_tiles(tb)
                transpose(bf, j, tb)
                g = sg * _SG + j
                for dd in range(n_dd):
                    tile_copy(g, tb, dd).start()
            if load:
                idx_copy(sg + 2, bf).start()

        idx_copy(s_base + 0, 0).start()
        idx_copy(s_base + 1, 1).start()
        idx_copy(0, 0).wait()
        gather_copy(0).start()
        body(s_base + 0, 0, launch=True, load=True, first=True)

        def steady(s, carry):
            sg = s_base + 1 + s * 2
            body(sg, 1, launch=True, load=True, first=False)
            body(sg + 1, 0, launch=True, load=True, first=False)
            return carry

        lax.fori_loop(0, n_steady // 2, steady, 0)

        for t in range(tail_start, s_w):
            body(s_base + t, t % 2,
                 launch=(t + 1 < s_w), load=(t + 2 < s_w), first=False)
        for tb in range(2):
            drain_tiles(tb)

    return k(idx_flat, weight)


def kernel(idx, weight):
    b, h = idx.shape
    v, d = weight.shape
    idx_flat = idx.T.reshape(b * h).astype(jnp.int32)
    out5 = _sc_embedding_gather(idx_flat, weight, b=b, h=h, d=d)
    return out5.transpose(2, 4, 0, 1, 3).reshape(b, h, d)
